# baseline (speedup 1.0000x reference)
import functools
import math

import jax
import jax.numpy as jnp
from jax import lax
from jax.experimental import pallas as pl
from jax.experimental.pallas import tpu as pltpu
from jax.experimental.pallas import tpu_sc as plsc

D_MODEL = 32
SCALE = math.sqrt(D_MODEL)

_info = plsc.get_sparse_core_info()
NC, NS, L = _info.num_cores, _info.num_subcores, _info.num_lanes
NW = NC * NS

GW = 128
JB = 8
JH = 4
NG = 4


def _emb_kernel(nblk, xT2, table, out5, idx_v, rows_v, tbuf, *sems):
  gsem = sems[:NG]
  wsem = sems[NG:]
  wid = lax.axis_index("s") * NC + lax.axis_index("c")
  q0 = wid * (nblk * JB)

  def stage_idx(blk):
    bi = lax.rem(blk, 2)
    base = q0 + blk * JB
    pltpu.sync_copy(xT2.at[pl.ds(pl.multiple_of(base, 8), JB)],
                    idx_v.at[bi])

  def fire_gathers(blk, h, gb):
    bi = lax.rem(blk, 2)
    for jj in range(JH):
      pltpu.async_copy(table.at[idx_v.at[bi].at[h * JH + jj]],
                       rows_v.at[gb].at[pl.ds(jj * GW, GW)], gsem[gb])

  def wait_gathers(gb):
    pltpu.make_async_copy(table.at[pl.ds(0, JH * GW)], rows_v.at[gb],
                          gsem[gb]).wait()

  def t_bq(blk, h, jj):
    q = q0 + blk * JB + h * JH + jj
    return q >> 7, q & 127

  def transpose_scale(gb, tbb):
    rv = rows_v.at[gb]
    tb = tbuf.at[tbb]

    for jj in range(JH):
      iota = lax.iota(jnp.int32, L)
      rows0 = [iota + (jj * GW + bg * L) for bg in range(GW // L)]

      @plsc.parallel_loop(0, D_MODEL, step=1, unroll=4)
      def _(d):
        col = jnp.full((L,), d, dtype=jnp.int32)
        for bg in range(GW // L):
          v = plsc.load_gather(rv, [rows0[bg], col])
          tb[jj * D_MODEL + d, pl.ds(bg * L, L)] = v * SCALE

  def write_copies(blk, h, tbb):
    copies = []
    for jj in range(JH):
      t, bq = t_bq(blk, h, jj)
      copies += [
          pltpu.make_async_copy(
              tbuf.at[tbb].at[pl.ds(jj * D_MODEL + dq * 8, 8)],
              out5.at[t, dq, bq], wsem[tbb])
          for dq in range(D_MODEL // 8)
      ]
    return copies

  stage_idx(0)
  fire_gathers(0, 0, 0)
  fire_gathers(0, 1, 1)

  def outer(bp, _):
    for hh in range(4):
      blk = 2 * bp + hh // 2
      h = hh % 2
      tbb = hh % 2
      if hh % 2 == 0:
        @pl.when(blk + 1 < nblk)
        def _():
          stage_idx(blk + 1)

      @pl.when(blk + 1 < nblk)
      def _():
        fire_gathers(blk + 1, h, (hh + 2) % NG)

      wait_gathers(hh)

      @pl.when(2 * blk + h >= 2)
      def _():
        for c in write_copies(blk - 1, h, tbb):
          c.wait()

      transpose_scale(hh, tbb)
      for c in write_copies(blk, h, tbb):
        c.start()
    return 0

  lax.fori_loop(0, nblk // 2, outer, 0)
  for c in write_copies(nblk - 1, 0, 0):
    c.wait()
  for c in write_copies(nblk - 1, 1, 1):
    c.wait()


def kernel(x, emb_weight):
  B = x.shape[0] * x.shape[1]
  T = x.shape[1]
  NB = x.shape[0]
  Q = B // GW
  assert Q % (NW * JB * 2) == 0 and NB % GW == 0 and T % 8 == 0
  nblk = Q // (NW * JB)
  xT2 = x.T.reshape(Q, GW).astype(jnp.int32)

  mesh = plsc.VectorSubcoreMesh(core_axis_name="c", subcore_axis_name="s")
  run = pl.kernel(
      functools.partial(_emb_kernel, nblk),
      out_type=jax.ShapeDtypeStruct(
          (T, D_MODEL // 8, NB // GW, 8, GW), jnp.float32),
      mesh=mesh,
      scratch_types=[
          pltpu.VMEM((2, JB, GW), jnp.int32),
          pltpu.VMEM((NG, JH * GW, D_MODEL), jnp.float32),
          pltpu.VMEM((2, JH * D_MODEL, GW), jnp.float32),
      ] + [pltpu.SemaphoreType.DMA] * (NG + 2),
      compiler_params=pltpu.CompilerParams(use_tc_tiling_on_sc=False,
                                           needs_layout_passes=False),
  )
  out5 = run(xT2, emb_weight)
  return out5.transpose(2, 4, 0, 1, 3).reshape(NB, T, D_MODEL)


# speedup vs baseline: 2.2321x; 2.2321x over previous
import functools
import math

import jax
import jax.numpy as jnp
from jax import lax
from jax.experimental import pallas as pl
from jax.experimental.pallas import tpu as pltpu
from jax.experimental.pallas import tpu_sc as plsc

D_MODEL = 32
SCALE = math.sqrt(D_MODEL)

_info = plsc.get_sparse_core_info()
NC, NS, L = _info.num_cores, _info.num_subcores, _info.num_lanes
NW = NC * NS

GW = 128
JB = 8
JH = 4
NG = 4
TBP = 129


def _emb_kernel(nblk, xT2, table, out5, idx_v, rows_v, tbuf, *sems):
  gsem = sems[:NG]
  wsem = sems[NG:]
  wid = lax.axis_index("s") * NC + lax.axis_index("c")
  q0 = wid * (nblk * JB)

  def stage_idx(blk):
    bi = lax.rem(blk, 2)
    base = q0 + blk * JB
    pltpu.sync_copy(xT2.at[pl.ds(pl.multiple_of(base, 8), JB)],
                    idx_v.at[bi])

  def fire_gathers(blk, h, gb):
    bi = lax.rem(blk, 2)
    for jj in range(JH):
      pltpu.async_copy(table.at[idx_v.at[bi].at[h * JH + jj]],
                       rows_v.at[gb].at[pl.ds(jj * GW, GW)], gsem[gb])

  def wait_gathers(gb):
    pltpu.make_async_copy(table.at[pl.ds(0, JH * GW)], rows_v.at[gb],
                          gsem[gb]).wait()

  def t_bq(blk, h, jj):
    q = q0 + blk * JB + h * JH + jj
    return q >> 7, q & 127

  def transpose_scale(gb, tbb):
    rv = rows_v.at[gb]
    tb = tbuf.at[tbb]

    for jj in range(JH):
      iota = lax.iota(jnp.int32, L)
      rows_lo = iota + jj * D_MODEL
      rows_hi = rows_lo + L

      @plsc.parallel_loop(0, GW, step=1, unroll=4)
      def _(r):
        colr = jnp.full((L,), r, dtype=jnp.int32)
        v0 = rv[jj * GW + r, pl.ds(0, L)]
        plsc.store_scatter(tb, [rows_lo, colr], v0 * SCALE)
        v1 = rv[jj * GW + r, pl.ds(L, L)]
        plsc.store_scatter(tb, [rows_hi, colr], v1 * SCALE)

  def write_copies(blk, h, tbb):
    copies = []
    for jj in range(JH):
      t, bq = t_bq(blk, h, jj)
      copies += [
          pltpu.make_async_copy(
              tbuf.at[tbb].at[pl.ds(jj * D_MODEL + dq * 8, 8), pl.ds(0, GW)],
              out5.at[t, dq, bq], wsem[tbb])
          for dq in range(D_MODEL // 8)
      ]
    return copies

  stage_idx(0)
  fire_gathers(0, 0, 0)
  fire_gathers(0, 1, 1)

  def outer(bp, _):
    for hh in range(4):
      blk = 2 * bp + hh // 2
      h = hh % 2
      tbb = hh % 2
      if hh % 2 == 0:
        @pl.when(blk + 1 < nblk)
        def _():
          stage_idx(blk + 1)

      @pl.when(blk + 1 < nblk)
      def _():
        fire_gathers(blk + 1, h, (hh + 2) % NG)

      wait_gathers(hh)

      @pl.when(2 * blk + h >= 2)
      def _():
        for c in write_copies(blk - 1, h, tbb):
          c.wait()

      transpose_scale(hh, tbb)
      for c in write_copies(blk, h, tbb):
        c.start()
    return 0

  lax.fori_loop(0, nblk // 2, outer, 0)
  for c in write_copies(nblk - 1, 0, 0):
    c.wait()
  for c in write_copies(nblk - 1, 1, 1):
    c.wait()


def kernel(x, emb_weight):
  B = x.shape[0] * x.shape[1]
  T = x.shape[1]
  NB = x.shape[0]
  Q = B // GW
  assert Q % (NW * JB * 2) == 0 and NB % GW == 0 and T % 8 == 0
  nblk = Q // (NW * JB)
  xT2 = x.T.reshape(Q, GW).astype(jnp.int32)

  mesh = plsc.VectorSubcoreMesh(core_axis_name="c", subcore_axis_name="s")
  run = pl.kernel(
      functools.partial(_emb_kernel, nblk),
      out_type=jax.ShapeDtypeStruct(
          (T, D_MODEL // 8, NB // GW, 8, GW), jnp.float32),
      mesh=mesh,
      scratch_types=[
          pltpu.VMEM((2, JB, GW), jnp.int32),
          pltpu.VMEM((NG, JH * GW, D_MODEL), jnp.float32),
          pltpu.VMEM((2, JH * D_MODEL, TBP), jnp.float32),
      ] + [pltpu.SemaphoreType.DMA] * (NG + 2),
      compiler_params=pltpu.CompilerParams(use_tc_tiling_on_sc=False,
                                           needs_layout_passes=False),
  )
  out5 = run(xT2, emb_weight)
  return out5.transpose(2, 4, 0, 1, 3).reshape(NB, T, D_MODEL)


# speedup vs baseline: 2.2477x; 1.0070x over previous
import functools
import math

import jax
import jax.numpy as jnp
from jax import lax
from jax.experimental import pallas as pl
from jax.experimental.pallas import tpu as pltpu
from jax.experimental.pallas import tpu_sc as plsc

D_MODEL = 32
SCALE = math.sqrt(D_MODEL)

_info = plsc.get_sparse_core_info()
NC, NS, L = _info.num_cores, _info.num_subcores, _info.num_lanes
NW = NC * NS

GW = 128
JB = 8
JH = 4
NG = 4
TBP = 129


def _emb_kernel(nblk, xT2, table, out5, idx_v, rows_v, tbuf, *sems):
  gsem = sems[:NG]
  wsem = sems[NG:]
  wid = lax.axis_index("s") * NC + lax.axis_index("c")
  q0 = wid * (nblk * JB)

  def stage_idx(blk):
    bi = lax.rem(blk, 2)
    q = q0 + blk * JB
    t = q >> 7
    pltpu.sync_copy(
        xT2.at[t >> 3, pl.ds(pl.multiple_of(q & 127, 8), JB), t & 7],
        idx_v.at[bi])

  def fire_gathers(blk, h, gb):
    bi = lax.rem(blk, 2)
    for jj in range(JH):
      pltpu.async_copy(table.at[idx_v.at[bi].at[h * JH + jj]],
                       rows_v.at[gb].at[pl.ds(jj * GW, GW)], gsem[gb])

  def wait_gathers(gb):
    pltpu.make_async_copy(table.at[pl.ds(0, JH * GW)], rows_v.at[gb],
                          gsem[gb]).wait()

  def t_bq(blk, h, jj):
    q = q0 + blk * JB + h * JH + jj
    return q >> 7, q & 127

  def transpose_scale(gb, tbb):
    rv = rows_v.at[gb]
    tb = tbuf.at[tbb]

    for jj in range(JH):
      iota = lax.iota(jnp.int32, L)
      rows_lo = iota + jj * D_MODEL
      rows_hi = rows_lo + L

      @plsc.parallel_loop(0, GW, step=1, unroll=4)
      def _(r):
        colr = jnp.full((L,), r, dtype=jnp.int32)
        v0 = rv[jj * GW + r, pl.ds(0, L)]
        plsc.store_scatter(tb, [rows_lo, colr], v0 * SCALE)
        v1 = rv[jj * GW + r, pl.ds(L, L)]
        plsc.store_scatter(tb, [rows_hi, colr], v1 * SCALE)

  def write_copies(blk, h, tbb):
    copies = []
    for jj in range(JH):
      t, bq = t_bq(blk, h, jj)
      copies += [
          pltpu.make_async_copy(
              tbuf.at[tbb].at[pl.ds(jj * D_MODEL + dq * 8, 8), pl.ds(0, GW)],
              out5.at[t, dq, bq], wsem[tbb])
          for dq in range(D_MODEL // 8)
      ]
    return copies

  stage_idx(0)
  fire_gathers(0, 0, 0)
  fire_gathers(0, 1, 1)

  def outer(bp, _):
    for hh in range(4):
      blk = 2 * bp + hh // 2
      h = hh % 2
      tbb = hh % 2
      if hh % 2 == 0:
        @pl.when(blk + 1 < nblk)
        def _():
          stage_idx(blk + 1)

      @pl.when(blk + 1 < nblk)
      def _():
        fire_gathers(blk + 1, h, (hh + 2) % NG)

      wait_gathers(hh)

      @pl.when(2 * blk + h >= 2)
      def _():
        for c in write_copies(blk - 1, h, tbb):
          c.wait()

      transpose_scale(hh, tbb)
      for c in write_copies(blk, h, tbb):
        c.start()
    return 0

  lax.fori_loop(0, nblk // 2, outer, 0)
  for c in write_copies(nblk - 1, 0, 0):
    c.wait()
  for c in write_copies(nblk - 1, 1, 1):
    c.wait()


def kernel(x, emb_weight):
  B = x.shape[0] * x.shape[1]
  T = x.shape[1]
  NB = x.shape[0]
  Q = B // GW
  assert Q % (NW * JB * 2) == 0 and NB % GW == 0 and T % 8 == 0
  nblk = Q // (NW * JB)
  xT2 = x.T.reshape(T // 8, 8, NB // GW, GW).transpose(0, 2, 1, 3)
  xT2 = xT2.astype(jnp.int32)

  mesh = plsc.VectorSubcoreMesh(core_axis_name="c", subcore_axis_name="s")
  run = pl.kernel(
      functools.partial(_emb_kernel, nblk),
      out_type=jax.ShapeDtypeStruct(
          (T, D_MODEL // 8, NB // GW, 8, GW), jnp.float32),
      mesh=mesh,
      scratch_types=[
          pltpu.VMEM((2, JB, GW), jnp.int32),
          pltpu.VMEM((NG, JH * GW, D_MODEL), jnp.float32),
          pltpu.VMEM((2, JH * D_MODEL, TBP), jnp.float32),
      ] + [pltpu.SemaphoreType.DMA] * (NG + 2),
      compiler_params=pltpu.CompilerParams(use_tc_tiling_on_sc=False,
                                           needs_layout_passes=False),
  )
  out5 = run(xT2, emb_weight)
  return out5.transpose(2, 4, 0, 1, 3).reshape(NB, T, D_MODEL)


# speedup vs baseline: 2.2480x; 1.0001x over previous
import functools
import math

import jax
import jax.numpy as jnp
from jax import lax
from jax.experimental import pallas as pl
from jax.experimental.pallas import tpu as pltpu
from jax.experimental.pallas import tpu_sc as plsc

D_MODEL = 32
SCALE = math.sqrt(D_MODEL)

_info = plsc.get_sparse_core_info()
NC, NS, L = _info.num_cores, _info.num_subcores, _info.num_lanes
NW = NC * NS

GW = 128
JB = 8
JH = 4
NG = 4
TBP = 129


def _emb_kernel(nblk, xT2, table, out5, idx_v, rows_v, tbuf, *sems):
  gsem = sems[:NG]
  wsem = sems[NG:]
  wid = lax.axis_index("s") * NC + lax.axis_index("c")
  q0 = wid * (nblk * JB)

  def stage_idx(blk):
    bi = lax.rem(blk, 2)
    q = q0 + blk * JB
    t = q >> 7
    pltpu.sync_copy(
        xT2.at[t >> 3, pl.ds(pl.multiple_of(q & 127, 8), JB), t & 7],
        idx_v.at[bi])

  def fire_gathers(blk, h, gb):
    bi = lax.rem(blk, 2)
    for jj in range(JH):
      pltpu.async_copy(table.at[idx_v.at[bi].at[h * JH + jj]],
                       rows_v.at[gb].at[pl.ds(jj * GW, GW)], gsem[gb])

  def wait_gathers(gb):
    pltpu.make_async_copy(table.at[pl.ds(0, JH * GW)], rows_v.at[gb],
                          gsem[gb]).wait()

  def t_bq(blk, h, jj):
    q = q0 + blk * JB + h * JH + jj
    return q >> 7, q & 127

  def transpose_scale(gb, tbb):
    rv = rows_v.at[gb]
    tb = tbuf.at[tbb]

    for jj in range(JH):
      iota = lax.iota(jnp.int32, L)
      rows_lo = iota + jj * D_MODEL
      rows_hi = rows_lo + L

      @plsc.parallel_loop(0, GW, step=1, unroll=4)
      def _(r):
        colr = jnp.full((L,), r, dtype=jnp.int32)
        v0 = rv[jj * GW + r, pl.ds(0, L)]
        plsc.store_scatter(tb, [rows_lo, colr], v0 * SCALE)
        v1 = rv[jj * GW + r, pl.ds(L, L)]
        plsc.store_scatter(tb, [rows_hi, colr], v1 * SCALE)

  def write_copies(blk, h, tbb):
    copies = []
    for jj in range(JH):
      t, bq = t_bq(blk, h, jj)
      copies += [
          pltpu.make_async_copy(
              tbuf.at[tbb].at[pl.ds(jj * D_MODEL + dq * 8, 8), pl.ds(0, GW)],
              out5.at[t, dq, bq], wsem[tbb])
          for dq in range(D_MODEL // 8)
      ]
    return copies

  stage_idx(0)
  fire_gathers(0, 0, 0)
  fire_gathers(0, 1, 1)

  def outer(bp, _):
    for hh in range(4):
      blk = 2 * bp + hh // 2
      h = hh % 2
      tbb = hh % 2
      if hh % 2 == 0:
        @pl.when(blk + 1 < nblk)
        def _():
          stage_idx(blk + 1)

      @pl.when(blk + 1 < nblk)
      def _():
        fire_gathers(blk + 1, h, (hh + 2) % NG)

      wait_gathers(hh)

      @pl.when(2 * blk + h >= 2)
      def _():
        for c in write_copies(blk - 1, h, tbb):
          c.wait()

      transpose_scale(hh, tbb)
      for c in write_copies(blk, h, tbb):
        c.start()
    return 0

  lax.fori_loop(0, nblk // 2, outer, 0)
  for c in write_copies(nblk - 1, 0, 0):
    c.wait()
  for c in write_copies(nblk - 1, 1, 1):
    c.wait()


def kernel(x, emb_weight):
  B = x.shape[0] * x.shape[1]
  T = x.shape[1]
  NB = x.shape[0]
  Q = B // GW
  assert Q % (NW * JB * 2) == 0 and NB % GW == 0 and T % 8 == 0
  nblk = Q // (NW * JB)
  xT2 = x.T.reshape(T // 8, 8, NB // GW, GW).transpose(0, 2, 1, 3)
  xT2 = xT2.astype(jnp.int32)

  mesh = plsc.VectorSubcoreMesh(core_axis_name="c", subcore_axis_name="s")
  run = pl.kernel(
      functools.partial(_emb_kernel, nblk),
      out_type=jax.ShapeDtypeStruct(
          (T, D_MODEL // 8, NB // GW, 8, GW), jnp.float32),
      mesh=mesh,
      scratch_types=[
          pltpu.VMEM((2, JB, GW), jnp.int32),
          pltpu.VMEM((NG, JH * GW, D_MODEL), jnp.float32),
          pltpu.VMEM((2, JH * D_MODEL, TBP), jnp.float32),
      ] + [pltpu.SemaphoreType.DMA] * (NG + 2),
      compiler_params=pltpu.CompilerParams(use_tc_tiling_on_sc=False,
                                           needs_layout_passes=False),
  )
  wT = lax.optimization_barrier(emb_weight.T)
  out5 = run(xT2, wT.T)
  return out5.transpose(2, 4, 0, 1, 3).reshape(NB, T, D_MODEL)
